# Initial kernel scaffold; baseline (speedup 1.0000x reference)
import jax
import jax.numpy as jnp
from jax.experimental import pallas as pl


def kernel(x_nchw, conv1_w, conv1_b, conv1b_w, conv1b_b, conv2_w, conv2_b, conv2b_w, conv2b_b, conv3_w, conv3_b, fc1_w, fc1_b, fc2_w, fc2_b, fc3_w, fc3_b):
    raise NotImplementedError("write your pallas kernel here")


# speedup vs baseline: 1.9777x; 1.9777x over previous
import jax
import jax.numpy as jnp
from jax.experimental import pallas as pl
from jax.experimental.pallas import tpu as pltpu

B = 8


def _conv5(xw_ref, w_ref, b_ref, B_, H, W, C5, Cout):
    M = B_ * H * W
    acc = None
    for kh in range(5):
        slab = xw_ref[:, kh:kh + H, :, :].reshape(M, C5)
        d = jnp.dot(slab, w_ref[kh], preferred_element_type=jnp.float32)
        acc = d if acc is None else acc + d
    return jnp.maximum(acc + b_ref[...], 0.0)


def _scatter(y4, xw_ref, W, C):
    yp = jnp.pad(y4, ((0, 0), (2, 2), (2, 2), (0, 0)))
    for kw in range(5):
        xw_ref[:, :, :, kw * C:(kw + 1) * C] = yp[:, :, kw:kw + W, :]


def _pool(y4):
    b, h, w, c = y4.shape
    m = y4.reshape(b, h, w // 2, 2, c)
    m = jnp.maximum(m[:, :, :, 0, :], m[:, :, :, 1, :])
    m = m.reshape(b, h // 2, 2, w // 2, c)
    return jnp.maximum(m[:, :, 0], m[:, :, 1])


def _fused_kernel(x_ref,
                  w1, b1, w1b, b1b, w2, b2, w2b, b2b, w3, b3,
                  fw1, fb1, fw2, fb2, fw3, fb3,
                  o_ref,
                  xwA, xwB, xwC, xwD, xwE):
    bf16 = jnp.bfloat16
    x4 = x_ref[...].reshape(B, 32, 32, 3).astype(bf16)
    _scatter(x4, xwA, 32, 3)
    y = _conv5(xwA, w1, b1, B, 32, 32, 15, 32)
    _scatter(y.astype(bf16).reshape(B, 32, 32, 32), xwB, 32, 32)
    y = _conv5(xwB, w1b, b1b, B, 32, 32, 160, 64)
    p = _pool(y.reshape(B, 32, 32, 64))
    _scatter(p.astype(bf16), xwC, 16, 64)
    y = _conv5(xwC, w2, b2, B, 16, 16, 320, 128)
    _scatter(y.astype(bf16).reshape(B, 16, 16, 128), xwD, 16, 128)
    y = _conv5(xwD, w2b, b2b, B, 16, 16, 640, 64)
    p = _pool(y.reshape(B, 16, 16, 64))
    _scatter(p.astype(bf16), xwE, 8, 64)
    y = _conv5(xwE, w3, b3, B, 8, 8, 320, 16)
    p = _pool(y.reshape(B, 8, 8, 16))
    flat = jnp.concatenate(
        [p[:, h, w, :] for h in range(4) for w in range(4)], axis=1)
    h1 = jnp.dot(flat.astype(bf16), fw1[...],
                 preferred_element_type=jnp.float32) + fb1[...]
    h1 = jnp.maximum(h1, 0.0)
    h2 = jnp.dot(h1.astype(bf16), fw2[...],
                 preferred_element_type=jnp.float32) + fb2[...]
    h2 = jnp.maximum(h2, 0.0)
    o_ref[...] = jnp.dot(h2.astype(bf16), fw3[...],
                         preferred_element_type=jnp.float32) + fb3[...]


def kernel(x_nchw, conv1_w, conv1_b, conv1b_w, conv1b_b, conv2_w, conv2_b,
           conv2b_w, conv2b_b, conv3_w, conv3_b, fc1_w, fc1_b, fc2_w, fc2_b,
           fc3_w, fc3_b):
    N = x_nchw.shape[0]
    bf16 = jnp.bfloat16
    x = jnp.transpose(x_nchw, (0, 2, 3, 1)).reshape(N, 32, 96)

    def wk(w, cin):
        return w.reshape(5, 5 * cin, -1).astype(bf16)

    wargs = [wk(conv1_w, 3), conv1_b, wk(conv1b_w, 32), conv1b_b,
             wk(conv2_w, 64), conv2_b, wk(conv2b_w, 128), conv2b_b,
             wk(conv3_w, 64), conv3_b,
             fc1_w.astype(bf16), fc1_b, fc2_w.astype(bf16), fc2_b,
             fc3_w.astype(bf16), fc3_b]

    in_specs = [pl.BlockSpec((B, 32, 96), lambda n: (n, 0, 0))]
    in_specs += [pl.BlockSpec(a.shape, lambda n, nd=a.ndim: (0,) * nd)
                 for a in wargs]

    out = pl.pallas_call(
        _fused_kernel,
        out_shape=jax.ShapeDtypeStruct((N, 10), jnp.float32),
        grid=(N // B,),
        in_specs=in_specs,
        out_specs=pl.BlockSpec((B, 10), lambda n: (n, 0)),
        scratch_shapes=[
            pltpu.VMEM((B, 36, 32, 15), bf16),
            pltpu.VMEM((B, 36, 32, 160), bf16),
            pltpu.VMEM((B, 20, 16, 320), bf16),
            pltpu.VMEM((B, 20, 16, 640), bf16),
            pltpu.VMEM((B, 12, 8, 320), bf16),
        ],
        compiler_params=pltpu.CompilerParams(
            dimension_semantics=("parallel",),
            vmem_limit_bytes=100 * 1024 * 1024),
    )(x, *wargs)
    return out


# speedup vs baseline: 2.4920x; 1.2601x over previous
import jax
import jax.numpy as jnp
from jax.experimental import pallas as pl
from jax.experimental.pallas import tpu as pltpu

B = 8


def _conv_chunk(xw_ref, w_ref, b_ref, g0, GI, H, W, C5, Cout):
    m = GI * H * W
    acc = None
    for kh in range(5):
        slab = xw_ref[g0:g0 + GI, kh:kh + H, :, :].reshape(m, C5)
        d = jnp.dot(slab, w_ref[kh], preferred_element_type=jnp.float32)
        acc = d if acc is None else acc + d
    return jnp.maximum(acc + b_ref[...], 0.0)


def _scatter(y4, xw_ref, g0, W, C):
    gi = y4.shape[0]
    yp = jnp.pad(y4, ((0, 0), (2, 2), (2, 2), (0, 0)))
    for kw in range(5):
        xw_ref[g0:g0 + gi, :, :, kw * C:(kw + 1) * C] = yp[:, :, kw:kw + W, :]


def _pool(y4):
    b, h, w, c = y4.shape
    m = y4.reshape(b, h, w // 2, 2, c)
    m = jnp.maximum(m[:, :, :, 0, :], m[:, :, :, 1, :])
    m = m.reshape(b, h // 2, 2, w // 2, c)
    return jnp.maximum(m[:, :, 0], m[:, :, 1])


def _layer(xw_in, w, b, xw_out, GI, H, W, C5, Cout, pool):
    bf16 = jnp.bfloat16
    for g0 in range(0, B, GI):
        y = _conv_chunk(xw_in, w, b, g0, GI, H, W, C5, Cout)
        y4 = y.reshape(GI, H, W, Cout)
        if pool:
            y4 = _pool(y4)
        s = 2 if pool else 1
        _scatter(y4.astype(bf16), xw_out, g0, W // s, Cout)


def _fused_kernel(x_ref,
                  w1, b1, w1b, b1b, w2, b2, w2b, b2b, w3, b3,
                  fw1, fb1, fw2, fb2, fw3, fb3,
                  o_ref,
                  xwA, xwB, xwC, xwD, xwE):
    bf16 = jnp.bfloat16
    x4 = x_ref[...].reshape(B, 32, 32, 3).astype(bf16)
    _scatter(x4, xwA, 0, 32, 3)
    _layer(xwA, w1, b1, xwB, 2, 32, 32, 15, 32, False)
    _layer(xwB, w1b, b1b, xwC, 1, 32, 32, 160, 64, True)
    _layer(xwC, w2, b2, xwD, 2, 16, 16, 320, 128, False)
    _layer(xwD, w2b, b2b, xwE, 2, 16, 16, 640, 64, True)
    y = _conv_chunk(xwE, w3, b3, 0, B, 8, 8, 320, 16)
    p = _pool(y.reshape(B, 8, 8, 16))
    flat = jnp.concatenate(
        [p[:, h, w, :] for h in range(4) for w in range(4)], axis=1)
    h1 = jnp.dot(flat.astype(bf16), fw1[...],
                 preferred_element_type=jnp.float32) + fb1[...]
    h1 = jnp.maximum(h1, 0.0)
    h2 = jnp.dot(h1.astype(bf16), fw2[...],
                 preferred_element_type=jnp.float32) + fb2[...]
    h2 = jnp.maximum(h2, 0.0)
    o_ref[...] = jnp.dot(h2.astype(bf16), fw3[...],
                         preferred_element_type=jnp.float32) + fb3[...]


def kernel(x_nchw, conv1_w, conv1_b, conv1b_w, conv1b_b, conv2_w, conv2_b,
           conv2b_w, conv2b_b, conv3_w, conv3_b, fc1_w, fc1_b, fc2_w, fc2_b,
           fc3_w, fc3_b):
    N = x_nchw.shape[0]
    bf16 = jnp.bfloat16
    x = jnp.transpose(x_nchw, (0, 2, 3, 1)).reshape(N, 32, 96)

    def wk(w, cin):
        return w.reshape(5, 5 * cin, -1).astype(bf16)

    wargs = [wk(conv1_w, 3), conv1_b, wk(conv1b_w, 32), conv1b_b,
             wk(conv2_w, 64), conv2_b, wk(conv2b_w, 128), conv2b_b,
             wk(conv3_w, 64), conv3_b,
             fc1_w.astype(bf16), fc1_b, fc2_w.astype(bf16), fc2_b,
             fc3_w.astype(bf16), fc3_b]

    in_specs = [pl.BlockSpec((B, 32, 96), lambda n: (n, 0, 0))]
    in_specs += [pl.BlockSpec(a.shape, lambda n, nd=a.ndim: (0,) * nd)
                 for a in wargs]

    out = pl.pallas_call(
        _fused_kernel,
        out_shape=jax.ShapeDtypeStruct((N, 10), jnp.float32),
        grid=(N // B,),
        in_specs=in_specs,
        out_specs=pl.BlockSpec((B, 10), lambda n: (n, 0)),
        scratch_shapes=[
            pltpu.VMEM((B, 36, 32, 15), bf16),
            pltpu.VMEM((B, 36, 32, 160), bf16),
            pltpu.VMEM((B, 20, 16, 320), bf16),
            pltpu.VMEM((B, 20, 16, 640), bf16),
            pltpu.VMEM((B, 12, 8, 320), bf16),
        ],
        compiler_params=pltpu.CompilerParams(
            dimension_semantics=("parallel",),
            vmem_limit_bytes=100 * 1024 * 1024),
    )(x, *wargs)
    return out
